# baseline (speedup 1.0000x reference)
import jax
import jax.numpy as jnp
from jax import lax
from jax.experimental import pallas as pl
from jax.experimental.pallas import tpu as pltpu
from jax.experimental.pallas import tpu_sc as plsc

TOTAL_INPUT_BITS = 2048
NUM_NEURONS = 4096
N_BITS = 14
BATCH = 512
MEM_SIZE = 2 ** N_BITS

NB = 512

NUM_WORKERS = 32
TOTAL_LOOKUPS = BATCH * NUM_NEURONS
PER_WORKER = TOTAL_LOOKUPS // NUM_WORKERS
CHUNK = 2048
SUB = 128
NSUB = CHUNK // SUB
NCHUNK = PER_WORKER // CHUNK


def _addr_kernel(bits_ref, conn_ref, out_ref):
    conn = conn_ref[...]
    cvals = lax.broadcasted_iota(jnp.int32, (TOTAL_INPUT_BITS, NB), 0)
    wlo = jnp.zeros((TOTAL_INPUT_BITS, NB), jnp.int32)
    whi = jnp.zeros((TOTAL_INPUT_BITS, NB), jnp.int32)
    for i in range(N_BITS):
        eq = cvals == conn[:, i][None, :]
        if i < 7:
            wlo = wlo + jnp.where(eq, jnp.int32(1 << i), jnp.int32(0))
        else:
            whi = whi + jnp.where(eq, jnp.int32(1 << (i - 7)), jnp.int32(0))
    bits = bits_ref[...]
    lo = jnp.dot(bits, wlo.astype(jnp.bfloat16),
                 preferred_element_type=jnp.float32)
    hi = jnp.dot(bits, whi.astype(jnp.bfloat16),
                 preferred_element_type=jnp.float32)
    addr = (lo + 128.0 * hi).astype(jnp.int32)
    nb = pl.program_id(0)
    neuron = nb * NB + lax.broadcasted_iota(jnp.int32, (BATCH, NB), 1)
    out_ref[...] = addr + neuron * MEM_SIZE


def _addresses(bits_bf16, connections):
    return pl.pallas_call(
        _addr_kernel,
        grid=(NUM_NEURONS // NB,),
        in_specs=[
            pl.BlockSpec((BATCH, TOTAL_INPUT_BITS), lambda i: (0, 0)),
            pl.BlockSpec((NB, N_BITS), lambda i: (i, 0)),
        ],
        out_specs=pl.BlockSpec((BATCH, NB), lambda i: (0, i)),
        out_shape=jax.ShapeDtypeStruct((BATCH, NUM_NEURONS), jnp.int32),
    )(bits_bf16, connections)


def _sc_body(mem_hbm, idx_hbm, out_hbm,
             raw0, raw1, row0, row1, vals0, vals1, res_v, sem0, sem1):
    wid = lax.axis_index("s") * 2 + lax.axis_index("c")
    base = wid * (PER_WORKER // SUB)
    raws = (raw0, raw1)
    rows = (row0, row1)
    vals = (vals0, vals1)
    sems = (sem0, sem1)

    def fire(ci, b):
        pltpu.sync_copy(
            idx_hbm.at[pl.ds((base + ci * NSUB) * SUB, CHUNK)], raws[b])
        for j in range(NSUB):
            for k in range(SUB // 16):
                p = j * SUB + k * 16
                rows[b][j, pl.ds(k * 16, 16)] = raws[b][pl.ds(p, 16)] >> 2
        for j in range(NSUB):
            pltpu.async_copy(mem_hbm.at[rows[b].at[j]],
                             vals[b].at[pl.ds(j * SUB, SUB)], sems[b])

    def drain_compare_store(ci, b):
        for j in range(NSUB):
            pltpu.make_async_copy(
                mem_hbm.at[pl.ds(0, SUB)],
                vals[b].at[pl.ds(j * SUB, SUB)], sems[b]).wait()
        for v in range(CHUNK // 16):
            x = vals[b][pl.ds(v * 16, 16)]
            sh = (raws[b][pl.ds(v * 16, 16)] & 3) << 3
            cell = (x >> sh) & 255
            res_v[pl.ds(v * 16, 16)] = jnp.where(
                cell == 1, jnp.int32(1), jnp.int32(0))
        pltpu.sync_copy(
            res_v, out_hbm.at[pl.ds((base + ci * NSUB) * SUB, CHUNK)])

    fire(0, 0)

    def chunk_body(ci, carry):
        @pl.when(ci + 1 < NCHUNK)
        def _next():
            @pl.when(lax.rem(ci + 1, 2) == 0)
            def _():
                fire(ci + 1, 0)

            @pl.when(lax.rem(ci + 1, 2) == 1)
            def _():
                fire(ci + 1, 1)

        @pl.when(lax.rem(ci, 2) == 0)
        def _even():
            drain_compare_store(ci, 0)

        @pl.when(lax.rem(ci, 2) == 1)
        def _odd():
            drain_compare_store(ci, 1)

        return carry

    lax.fori_loop(0, NCHUNK, chunk_body, 0)


def _sc_lookup(memw, idx1d):
    mesh = plsc.VectorSubcoreMesh(core_axis_name="c", subcore_axis_name="s")
    return pl.kernel(
        _sc_body,
        out_type=jax.ShapeDtypeStruct((TOTAL_LOOKUPS,), jnp.int32),
        mesh=mesh,
        scratch_types=[
            pltpu.VMEM((CHUNK,), jnp.int32),
            pltpu.VMEM((CHUNK,), jnp.int32),
            pltpu.VMEM((NSUB, SUB), jnp.int32),
            pltpu.VMEM((NSUB, SUB), jnp.int32),
            pltpu.VMEM((CHUNK,), jnp.int32),
            pltpu.VMEM((CHUNK,), jnp.int32),
            pltpu.VMEM((CHUNK,), jnp.int32),
            pltpu.SemaphoreType.DMA,
            pltpu.SemaphoreType.DMA,
        ],
    )(memw, idx1d)


def kernel(input_bits, connections, memory):
    bits_bf16 = input_bits.astype(jnp.bfloat16)
    flat_idx = _addresses(bits_bf16, connections)
    memw = lax.bitcast_convert_type(
        memory.astype(jnp.int8).reshape(NUM_NEURONS * MEM_SIZE // 4, 4),
        jnp.int32)
    res = _sc_lookup(memw, flat_idx.reshape(-1))
    return res.reshape(BATCH, NUM_NEURONS).astype(bool)


# speedup vs baseline: 1.6496x; 1.6496x over previous
import jax
import jax.numpy as jnp
from jax import lax
from jax.experimental import pallas as pl
from jax.experimental.pallas import tpu as pltpu
from jax.experimental.pallas import tpu_sc as plsc

TOTAL_INPUT_BITS = 2048
NUM_NEURONS = 4096
N_BITS = 14
BATCH = 512
MEM_SIZE = 2 ** N_BITS

NB = 512

NUM_WORKERS = 32
TOTAL_LOOKUPS = BATCH * NUM_NEURONS
PER_WORKER = TOTAL_LOOKUPS // NUM_WORKERS
CHUNK = 2048
SUB = 128
NSUB = CHUNK // SUB
NCHUNK = PER_WORKER // CHUNK


def _addr_kernel(bits_ref, conn_ref, out_ref):
    conn = conn_ref[...]
    cvals = lax.broadcasted_iota(jnp.int32, (TOTAL_INPUT_BITS, NB), 0)
    wlo = jnp.zeros((TOTAL_INPUT_BITS, NB), jnp.int32)
    whi = jnp.zeros((TOTAL_INPUT_BITS, NB), jnp.int32)
    for i in range(N_BITS):
        eq = cvals == conn[:, i][None, :]
        if i < 7:
            wlo = wlo + jnp.where(eq, jnp.int32(1 << i), jnp.int32(0))
        else:
            whi = whi + jnp.where(eq, jnp.int32(1 << (i - 7)), jnp.int32(0))
    bits = bits_ref[...]
    lo = jnp.dot(bits, wlo.astype(jnp.bfloat16),
                 preferred_element_type=jnp.float32)
    hi = jnp.dot(bits, whi.astype(jnp.bfloat16),
                 preferred_element_type=jnp.float32)
    addr = (lo + 128.0 * hi).astype(jnp.int32)
    nb = pl.program_id(0)
    neuron = nb * NB + lax.broadcasted_iota(jnp.int32, (BATCH, NB), 1)
    out_ref[...] = addr + neuron * MEM_SIZE


def _addresses(bits_bf16, connections):
    return pl.pallas_call(
        _addr_kernel,
        grid=(NUM_NEURONS // NB,),
        in_specs=[
            pl.BlockSpec((BATCH, TOTAL_INPUT_BITS), lambda i: (0, 0)),
            pl.BlockSpec((NB, N_BITS), lambda i: (i, 0)),
        ],
        out_specs=pl.BlockSpec((BATCH, NB), lambda i: (0, i)),
        out_shape=jax.ShapeDtypeStruct((BATCH, NUM_NEURONS), jnp.int32),
    )(bits_bf16, connections)


def _sc_body(mem_hbm, idx_hbm, out_hbm,
             raw0, raw1, row0, row1, vals0, vals1, res_v, sem0, sem1):
    wid = lax.axis_index("s") * 2 + lax.axis_index("c")
    base = wid * (PER_WORKER // SUB)
    raws = (raw0, raw1)
    rows = (row0, row1)
    vals = (vals0, vals1)
    sems = (sem0, sem1)

    def fire(ci, b):
        pltpu.sync_copy(
            idx_hbm.at[pl.ds((base + ci * NSUB) * SUB, CHUNK)], raws[b])
        for j in range(NSUB):
            for k in range(SUB // 16):
                p = j * SUB + k * 16
                rows[b][j, pl.ds(k * 16, 16)] = raws[b][pl.ds(p, 16)] >> 2
        for j in range(NSUB):
            pltpu.async_copy(mem_hbm.at[rows[b].at[j]],
                             vals[b].at[pl.ds(j * SUB, SUB)], sems[b])

    def drain_compare_store(ci, b):
        for j in range(NSUB):
            pltpu.make_async_copy(
                mem_hbm.at[pl.ds(0, SUB)],
                vals[b].at[pl.ds(j * SUB, SUB)], sems[b]).wait()
        for v in range(CHUNK // 16):
            x = vals[b][pl.ds(v * 16, 16)]
            sh = (raws[b][pl.ds(v * 16, 16)] & 3) << 3
            cell = (x >> sh) & 255
            res_v[pl.ds(v * 16, 16)] = jnp.where(
                cell == 1, jnp.int32(1), jnp.int32(0))
        pltpu.sync_copy(
            res_v, out_hbm.at[pl.ds((base + ci * NSUB) * SUB, CHUNK)])

    fire(0, 0)

    def chunk_body(ci, carry):
        @pl.when(ci + 1 < NCHUNK)
        def _next():
            @pl.when(lax.rem(ci + 1, 2) == 0)
            def _():
                fire(ci + 1, 0)

            @pl.when(lax.rem(ci + 1, 2) == 1)
            def _():
                fire(ci + 1, 1)

        @pl.when(lax.rem(ci, 2) == 0)
        def _even():
            drain_compare_store(ci, 0)

        @pl.when(lax.rem(ci, 2) == 1)
        def _odd():
            drain_compare_store(ci, 1)

        return carry

    lax.fori_loop(0, NCHUNK, chunk_body, 0)


def _sc_lookup(memw, idx1d):
    mesh = plsc.VectorSubcoreMesh(core_axis_name="c", subcore_axis_name="s")
    return pl.kernel(
        _sc_body,
        out_type=jax.ShapeDtypeStruct((TOTAL_LOOKUPS,), jnp.int32),
        mesh=mesh,
        scratch_types=[
            pltpu.VMEM((CHUNK,), jnp.int32),
            pltpu.VMEM((CHUNK,), jnp.int32),
            pltpu.VMEM((NSUB, SUB), jnp.int32),
            pltpu.VMEM((NSUB, SUB), jnp.int32),
            pltpu.VMEM((CHUNK,), jnp.int32),
            pltpu.VMEM((CHUNK,), jnp.int32),
            pltpu.VMEM((CHUNK,), jnp.int32),
            pltpu.SemaphoreType.DMA,
            pltpu.SemaphoreType.DMA,
        ],
    )(memw, idx1d)


def kernel(input_bits, connections, memory):
    bits_bf16 = input_bits.astype(jnp.bfloat16)
    flat_idx = _addresses(bits_bf16, connections)
    memw = (memory[:, 0::4]
            | (memory[:, 1::4] << 8)
            | (memory[:, 2::4] << 16)
            | (memory[:, 3::4] << 24)).reshape(-1)
    res = _sc_lookup(memw, flat_idx.reshape(-1))
    return res.reshape(BATCH, NUM_NEURONS).astype(bool)


# speedup vs baseline: 48.9666x; 29.6835x over previous
import jax
import jax.numpy as jnp
from jax import lax
from jax.experimental import pallas as pl
from jax.experimental.pallas import tpu as pltpu
from jax.experimental.pallas import tpu_sc as plsc

TOTAL_INPUT_BITS = 2048
NUM_NEURONS = 4096
N_BITS = 14
BATCH = 512
MEM_SIZE = 2 ** N_BITS

NB = 512

NUM_WORKERS = 32
TOTAL_LOOKUPS = BATCH * NUM_NEURONS
PER_WORKER = TOTAL_LOOKUPS // NUM_WORKERS
CHUNK = 2048
SUB = 128
NSUB = CHUNK // SUB
NCHUNK = PER_WORKER // CHUNK


def _addr_kernel(bits_ref, conn_ref, out_ref):
    conn = conn_ref[...]
    cvals = lax.broadcasted_iota(jnp.int32, (TOTAL_INPUT_BITS, NB), 0)
    wlo = jnp.zeros((TOTAL_INPUT_BITS, NB), jnp.int32)
    whi = jnp.zeros((TOTAL_INPUT_BITS, NB), jnp.int32)
    for i in range(N_BITS):
        eq = cvals == conn[:, i][None, :]
        if i < 7:
            wlo = wlo + jnp.where(eq, jnp.int32(1 << i), jnp.int32(0))
        else:
            whi = whi + jnp.where(eq, jnp.int32(1 << (i - 7)), jnp.int32(0))
    bits = bits_ref[...]
    lo = jnp.dot(bits, wlo.astype(jnp.bfloat16),
                 preferred_element_type=jnp.float32)
    hi = jnp.dot(bits, whi.astype(jnp.bfloat16),
                 preferred_element_type=jnp.float32)
    addr = (lo + 128.0 * hi).astype(jnp.int32)
    nb = pl.program_id(0)
    neuron = nb * NB + lax.broadcasted_iota(jnp.int32, (BATCH, NB), 1)
    out_ref[...] = addr + neuron * MEM_SIZE


def _addresses(bits_bf16, connections):
    return pl.pallas_call(
        _addr_kernel,
        grid=(NUM_NEURONS // NB,),
        in_specs=[
            pl.BlockSpec((BATCH, TOTAL_INPUT_BITS), lambda i: (0, 0)),
            pl.BlockSpec((NB, N_BITS), lambda i: (i, 0)),
        ],
        out_specs=pl.BlockSpec((BATCH, NB), lambda i: (0, i)),
        out_shape=jax.ShapeDtypeStruct((BATCH, NUM_NEURONS), jnp.int32),
    )(bits_bf16, connections)


def _sc_body(mem_hbm, idx_hbm, out_hbm,
             row0, row1, row2, vals0, vals1, vals2, res_v,
             sem0, sem1, sem2):
    wid = lax.axis_index("s") * 2 + lax.axis_index("c")
    base = wid * (PER_WORKER // SUB)
    rows = (row0, row1, row2)
    vals = (vals0, vals1, vals2)
    sems = (sem0, sem1, sem2)

    def fire(ci, b):
        pltpu.sync_copy(idx_hbm.at[pl.ds(base + ci * NSUB, NSUB)], rows[b])
        for j in range(NSUB):
            pltpu.async_copy(mem_hbm.at[rows[b].at[j]],
                             vals[b].at[pl.ds(j * SUB, SUB)], sems[b])

    def drain_compare_store(ci, b):
        for j in range(NSUB):
            pltpu.make_async_copy(
                mem_hbm.at[pl.ds(0, SUB)],
                vals[b].at[pl.ds(j * SUB, SUB)], sems[b]).wait()
        for v in range(CHUNK // 16):
            x = vals[b][pl.ds(v * 16, 16)]
            res_v[pl.ds(v * 16, 16)] = jnp.where(
                x == 1, jnp.int32(1), jnp.int32(0))
        pltpu.sync_copy(
            res_v, out_hbm.at[pl.ds((base + ci * NSUB) * SUB, CHUNK)])

    fire(0, 0)
    fire(1, 1)

    def chunk_body(ci, carry):
        @pl.when(ci + 2 < NCHUNK)
        def _next():
            for k in range(3):
                @pl.when(lax.rem(ci + 2, 3) == k)
                def _(k=k):
                    fire(ci + 2, k)

        for k in range(3):
            @pl.when(lax.rem(ci, 3) == k)
            def _(k=k):
                drain_compare_store(ci, k)

        return carry

    lax.fori_loop(0, NCHUNK, chunk_body, 0)


def _sc_lookup(mem1d, idx2d):
    mesh = plsc.VectorSubcoreMesh(core_axis_name="c", subcore_axis_name="s")
    return pl.kernel(
        _sc_body,
        out_type=jax.ShapeDtypeStruct((TOTAL_LOOKUPS,), jnp.int32),
        mesh=mesh,
        scratch_types=[
            pltpu.VMEM((NSUB, SUB), jnp.int32),
            pltpu.VMEM((NSUB, SUB), jnp.int32),
            pltpu.VMEM((NSUB, SUB), jnp.int32),
            pltpu.VMEM((CHUNK,), jnp.int32),
            pltpu.VMEM((CHUNK,), jnp.int32),
            pltpu.VMEM((CHUNK,), jnp.int32),
            pltpu.VMEM((CHUNK,), jnp.int32),
            pltpu.SemaphoreType.DMA,
            pltpu.SemaphoreType.DMA,
            pltpu.SemaphoreType.DMA,
        ],
    )(mem1d, idx2d)


def kernel(input_bits, connections, memory):
    bits_bf16 = input_bits.astype(jnp.bfloat16)
    flat_idx = _addresses(bits_bf16, connections)
    mem1d = memory.reshape(-1)
    idx2d = flat_idx.reshape(TOTAL_LOOKUPS // SUB, SUB)
    res = _sc_lookup(mem1d, idx2d)
    return res.reshape(BATCH, NUM_NEURONS).astype(bool)
